# baseline (speedup 1.0000x reference)
import numpy as np
import jax
import jax.numpy as jnp
from jax import lax
from jax.experimental import pallas as pl
from jax.experimental.pallas import tpu as pltpu
from jax.sharding import Mesh, PartitionSpec as P

N = 1024
ED = 2048
K = 4
BIG = 1e30
H = 512
EB = 512
RB = 256
KB = 512
DB = 256


def _dotT(x, w):
    return lax.dot_general(x, w, (((1,), (1,)), ((), ())),
                           preferred_element_type=jnp.float32)


def _encode_kernel(a_ref, t_ref, v_ref, wa_ref, ba_ref, wt_ref, bt_ref,
                   wv_ref, bv_ref, ea_ref, et_ref, ev_ref,
                   eab_ref, etb_ref, evb_ref, sqa_ref, sqt_ref, sqv_ref):
    i = pl.program_id(0)
    ea = _dotT(a_ref[...], wa_ref[...]) + ba_ref[...]
    et = _dotT(t_ref[...], wt_ref[...]) + bt_ref[...]
    ev = _dotT(v_ref[...], wv_ref[...]) + bv_ref[...]
    ea_ref[...] = ea
    et_ref[...] = et
    ev_ref[...] = ev
    eab_ref[...] = ea.astype(jnp.bfloat16)
    etb_ref[...] = et.astype(jnp.bfloat16)
    evb_ref[...] = ev.astype(jnp.bfloat16)
    pa = jnp.sum(ea * ea, axis=1, keepdims=True)
    pt = jnp.sum(et * et, axis=1, keepdims=True)
    pv = jnp.sum(ev * ev, axis=1, keepdims=True)

    @pl.when(i == 0)
    def _():
        sqa_ref[...] = pa
        sqt_ref[...] = pt
        sqv_ref[...] = pv

    @pl.when(i > 0)
    def _():
        sqa_ref[...] += pa
        sqt_ref[...] += pt
        sqv_ref[...] += pv


def _affinity_kernel(xl_ref, xf_ref, sql_ref, sqf_ref, mrow_ref, mcol_ref,
                     a_ref):
    i = pl.program_id(0)
    x_blk = xl_ref[pl.ds(i * RB, RB), :]
    sq_blk = sql_ref[pl.ds(i * RB, RB), :]
    g = _dotT(x_blk, xf_ref[...])
    d2 = sq_blk + sqf_ref[...].T - 2.0 * g
    iota = lax.broadcasted_iota(jnp.int32, (RB, N), 1)
    jstars = []
    sims = []
    for s in range(K):
        excl = jnp.zeros((RB, N), jnp.bool_)
        for j in jstars:
            excl = excl | (iota == j)
        deff = jnp.where(excl, BIG, d2)
        m = jnp.min(deff, axis=1, keepdims=True)
        jstar = jnp.min(jnp.where(deff == m, iota, N), axis=1, keepdims=True)
        jstars.append(jstar)
        sims.append(1.0 / (1.0 + jnp.sqrt(jnp.maximum(m, 0.0) + 1e-12)))
    a_blk = jnp.zeros((RB, N), jnp.float32)
    for jstar, sim in zip(jstars, sims):
        a_blk = a_blk + jnp.where(iota == jstar, sim, 0.0)
    a_blk = a_blk * mrow_ref[...] * mcol_ref[pl.ds(i * RB, RB), :]
    a_ref[...] = a_blk.astype(jnp.bfloat16)


def _symnorm_kernel(aa_ref, at_ref, av_ref, qa_ref, qt_ref, qv_ref):
    iota = lax.broadcasted_iota(jnp.int32, (N, N), 1)
    eye = iota == lax.broadcasted_iota(jnp.int32, (N, N), 0)

    def pmat(a_ref):
        a = a_ref[...].astype(jnp.float32)
        a = jnp.maximum(a, a.T)
        s = jnp.where(eye, 2.0, a)
        dc = lax.rsqrt(jnp.sum(s, axis=1, keepdims=True) + 1e-12)
        return dc * s * dc.T

    pa = pmat(aa_ref)
    pt = pmat(at_ref)
    pv = pmat(av_ref)
    qa_ref[...] = (pt + pv).astype(jnp.bfloat16)
    qt_ref[...] = (pv + pa).astype(jnp.bfloat16)
    qv_ref[...] = (pa + pt).astype(jnp.bfloat16)


def _fold_kernel(w_ref, d_ref, wb_ref, db_ref, m_ref, b2_ref, dbf_ref):
    i = pl.program_id(0)

    @pl.when(i == 0)
    def _():
        dbf = d_ref[...].astype(jnp.bfloat16)
        dbf_ref[...] = dbf
        b2_ref[...] = lax.dot_general(
            wb_ref[...].astype(jnp.bfloat16), dbf, (((1,), (1,)), ((), ())),
            preferred_element_type=jnp.float32) + db_ref[...]

    m_ref[...] = lax.dot_general(
        w_ref[...].astype(jnp.bfloat16), dbf_ref[...],
        (((0,), (1,)), ((), ())),
        preferred_element_type=jnp.float32).astype(jnp.bfloat16)


def _head_kernel(e_ref, m_ref, y_ref):
    y_ref[...] = lax.dot_general(e_ref[...], m_ref[...],
                                 (((1,), (0,)), ((), ())),
                                 preferred_element_type=jnp.float32)


def _mix_kernel(yl_ref, yf_ref, q_ref, b2_ref, o_ref):
    mixed = lax.dot_general(q_ref[...], yf_ref[...].astype(jnp.bfloat16),
                            (((1,), (0,)), ((), ())),
                            preferred_element_type=jnp.float32)
    o_ref[...] = 0.5 * yl_ref[...] + 0.25 * mixed + b2_ref[...]


def _impl(a, t, v, mask, Wa_w, Wa_b, Wt_w, Wt_b, Wv_w, Wv_b,
          wa_w, wa_b, wt_w, wt_b, wv_w, wv_b,
          da_w, da_b, dt_w, dt_b, dv_w, dv_b):
    f32 = jnp.float32
    bf16 = jnp.bfloat16
    idx = lax.axis_index('x')
    mrow = mask.reshape(1, N)
    mcol = lax.dynamic_slice(mask, (idx * H,), (H,)).reshape(H, 1)

    full = lambda shape: pl.BlockSpec(shape, lambda i: (0, 0))
    rows = lambda b, w: pl.BlockSpec((b, w), lambda i: (i, 0))
    ncols = lambda b: pl.BlockSpec((N, b), lambda i: (0, i))
    hcols = lambda b: pl.BlockSpec((H, b), lambda i: (0, i))

    enc_out = pl.pallas_call(
        _encode_kernel,
        grid=(ED // EB,),
        in_specs=[full((N, 1024)), full((N, 768)), full((N, 512)),
                  rows(EB, 1024), pl.BlockSpec((1, EB), lambda i: (0, i)),
                  rows(EB, 768), pl.BlockSpec((1, EB), lambda i: (0, i)),
                  rows(EB, 512), pl.BlockSpec((1, EB), lambda i: (0, i))],
        out_specs=[ncols(EB)] * 6 + [full((N, 1))] * 3,
        out_shape=[jax.ShapeDtypeStruct((N, ED), f32)] * 3
        + [jax.ShapeDtypeStruct((N, ED), bf16)] * 3
        + [jax.ShapeDtypeStruct((N, 1), f32)] * 3,
    )
    ea, et, ev, eab, etb, evb, sqa, sqt, sqv = enc_out(
        a, t, v, Wa_w, Wa_b.reshape(1, -1), Wt_w, Wt_b.reshape(1, -1),
        Wv_w, Wv_b.reshape(1, -1))

    aff = pl.pallas_call(
        _affinity_kernel,
        grid=(H // RB,),
        in_specs=[full((H, ED)), full((N, ED)), full((H, 1)), full((N, 1)),
                  full((1, N)), full((H, 1))],
        out_specs=rows(RB, N),
        out_shape=jax.ShapeDtypeStruct((H, N), bf16),
    )

    def aslice(x):
        return lax.dynamic_slice(x, (idx * H, 0), (H, x.shape[1]))

    aa = aff(aslice(ea), ea, aslice(sqa), sqa, mrow, mcol)
    at = aff(aslice(et), et, aslice(sqt), sqt, mrow, mcol)
    av = aff(aslice(ev), ev, aslice(sqv), sqv, mrow, mcol)

    abatch = jnp.concatenate([aa, at, av], axis=0)
    ag = lax.all_gather(abatch, 'x', axis=0, tiled=True)
    aaf = jnp.concatenate([ag[0 * H:1 * H], ag[3 * H:4 * H]], axis=0)
    atf = jnp.concatenate([ag[1 * H:2 * H], ag[4 * H:5 * H]], axis=0)
    avf = jnp.concatenate([ag[2 * H:3 * H], ag[5 * H:6 * H]], axis=0)

    qaf, qtf, qvf = pl.pallas_call(
        _symnorm_kernel,
        out_shape=[jax.ShapeDtypeStruct((N, N), bf16)] * 3,
    )(aaf, atf, avf)

    def fold(w_loc, wb, d, db):
        dout = d.shape[0]
        return pl.pallas_call(
            _fold_kernel,
            grid=(ED // 2 // KB,),
            in_specs=[pl.BlockSpec((ED, KB), lambda i: (0, i)),
                      full((dout, ED)), full((1, ED)), full((1, dout))],
            out_specs=[rows(KB, dout), full((1, dout))],
            out_shape=[jax.ShapeDtypeStruct((ED // 2, dout), bf16),
                       jax.ShapeDtypeStruct((1, dout), f32)],
            scratch_shapes=[pltpu.VMEM((dout, ED), bf16)],
        )(w_loc, d, wb.reshape(1, -1), db.reshape(1, -1))

    ma, b2a = fold(wa_w, wa_b, da_w, da_b)
    mt, b2t = fold(wt_w, wt_b, dt_w, dt_b)
    mv, b2v = fold(wv_w, wv_b, dv_w, dv_b)

    def head(eb, m_loc):
        dout = m_loc.shape[1]
        e_half = lax.dynamic_slice(eb, (0, idx * (ED // 2)), (N, ED // 2))
        return pl.pallas_call(
            _head_kernel,
            grid=(dout // DB,),
            in_specs=[full((N, ED // 2)),
                      pl.BlockSpec((ED // 2, DB), lambda i: (0, i))],
            out_specs=ncols(DB),
            out_shape=jax.ShapeDtypeStruct((N, dout), f32),
        )(e_half, m_loc)

    ya_p = head(eab, ma)
    yt_p = head(etb, mt)
    yv_p = head(evb, mv)

    ybatch = lax.psum(jnp.concatenate([ya_p, yt_p, yv_p], axis=1), 'x')
    ya = ybatch[:, 0:1024]
    yt = ybatch[:, 1024:1792]
    yv = ybatch[:, 1792:2304]

    def mix(yfull, q_full, b2):
        dout = yfull.shape[1]
        y_loc = lax.dynamic_slice(yfull, (idx * H, 0), (H, dout))
        q_loc = lax.dynamic_slice(q_full, (idx * H, 0), (H, N))
        return pl.pallas_call(
            _mix_kernel,
            grid=(dout // DB,),
            in_specs=[hcols(DB), ncols(DB), full((H, N)),
                      pl.BlockSpec((1, DB), lambda i: (0, i))],
            out_specs=hcols(DB),
            out_shape=jax.ShapeDtypeStruct((H, dout), f32),
        )(y_loc, yfull, q_loc, b2)

    ra = mix(ya, qaf, b2a)
    rt = mix(yt, qtf, b2t)
    rv = mix(yv, qvf, b2v)
    return jnp.concatenate([ra, rt, rv], axis=1)


def kernel(a, t, v, mask, Wa_w, Wa_b, Wt_w, Wt_b, Wv_w, Wv_b,
           wa_w, wa_b, wt_w, wt_b, wv_w, wv_b,
           da_w, da_b, dt_w, dt_b, dv_w, dv_b):
    mesh = Mesh(np.array(jax.devices()[:2]), ('x',))
    rep = P(None, None)
    sharded = jax.shard_map(
        _impl, mesh=mesh,
        in_specs=(rep, rep, rep, P(None),
                  rep, P(None), rep, P(None), rep, P(None),
                  P(None, 'x'), P(None), P(None, 'x'), P(None),
                  P(None, 'x'), P(None),
                  rep, P(None), rep, P(None), rep, P(None)),
        out_specs=P('x', None),
        check_vma=False,
    )
    return sharded(a, t, v, mask, Wa_w, Wa_b, Wt_w, Wt_b, Wv_w, Wv_b,
                   wa_w, wa_b, wt_w, wt_b, wv_w, wv_b,
                   da_w, da_b, dt_w, dt_b, dv_w, dv_b)


# speedup vs baseline: 3.6419x; 3.6419x over previous
import jax
import jax.numpy as jnp
from jax import lax
from jax.experimental import pallas as pl
from jax.experimental.pallas import tpu as pltpu

N = 1024
ED = 2048
K = 4
BIG = 1e30
EB = 512
RB = 256
DB = 256


def _bdotT(x, w):
    return lax.dot_general(x, w, (((1,), (1,)), ((), ())),
                           preferred_element_type=jnp.float32)


def _encode_kernel(a_ref, t_ref, v_ref, wa_ref, ba_ref, wt_ref, bt_ref,
                   wv_ref, bv_ref, eab_ref, etb_ref, evb_ref,
                   sqa_ref, sqt_ref, sqv_ref):
    i = pl.program_id(0)
    ea = _bdotT(a_ref[...], wa_ref[...].astype(jnp.bfloat16)) + ba_ref[...]
    et = _bdotT(t_ref[...], wt_ref[...].astype(jnp.bfloat16)) + bt_ref[...]
    ev = _bdotT(v_ref[...], wv_ref[...].astype(jnp.bfloat16)) + bv_ref[...]
    eab_ref[...] = ea.astype(jnp.bfloat16)
    etb_ref[...] = et.astype(jnp.bfloat16)
    evb_ref[...] = ev.astype(jnp.bfloat16)
    pa = jnp.sum(ea * ea, axis=1, keepdims=True)
    pt = jnp.sum(et * et, axis=1, keepdims=True)
    pv = jnp.sum(ev * ev, axis=1, keepdims=True)

    @pl.when(i == 0)
    def _():
        sqa_ref[...] = pa
        sqt_ref[...] = pt
        sqv_ref[...] = pv

    @pl.when(i > 0)
    def _():
        sqa_ref[...] += pa
        sqt_ref[...] += pt
        sqv_ref[...] += pv


def _affinity_kernel(xa_ref, xt_ref, xv_ref, sqa_ref, sqt_ref, sqv_ref,
                     mrow_ref, mcol_ref, aa_ref, at_ref, av_ref):
    i = pl.program_id(0)
    iota = lax.broadcasted_iota(jnp.int32, (RB, N), 1)
    mfac = mrow_ref[...] * mcol_ref[pl.ds(i * RB, RB), :]

    def one(x_ref, sq_ref, a_ref):
        x_blk = x_ref[pl.ds(i * RB, RB), :]
        sq_blk = sq_ref[pl.ds(i * RB, RB), :]
        g = _bdotT(x_blk, x_ref[...])
        d2 = sq_blk + sq_ref[...].T - 2.0 * g
        jstars = []
        sims = []
        for s in range(K):
            excl = jnp.zeros((RB, N), jnp.bool_)
            for j in jstars:
                excl = excl | (iota == j)
            deff = jnp.where(excl, BIG, d2)
            m = jnp.min(deff, axis=1, keepdims=True)
            jstar = jnp.min(jnp.where(deff == m, iota, N), axis=1,
                            keepdims=True)
            jstars.append(jstar)
            sims.append(1.0 / (1.0 + jnp.sqrt(jnp.maximum(m, 0.0) + 1e-12)))
        a_blk = jnp.zeros((RB, N), jnp.float32)
        for jstar, sim in zip(jstars, sims):
            a_blk = a_blk + jnp.where(iota == jstar, sim, 0.0)
        a_ref[...] = (a_blk * mfac).astype(jnp.bfloat16)

    one(xa_ref, sqa_ref, aa_ref)
    one(xt_ref, sqt_ref, at_ref)
    one(xv_ref, sqv_ref, av_ref)


def _symnorm_kernel(aa_ref, at_ref, av_ref, qa_ref, qt_ref, qv_ref):
    iota = lax.broadcasted_iota(jnp.int32, (N, N), 1)
    eye = iota == lax.broadcasted_iota(jnp.int32, (N, N), 0)

    def pmat(a_ref):
        a = a_ref[...].astype(jnp.float32)
        a = jnp.maximum(a, a.T)
        s = jnp.where(eye, 2.0, a)
        dc = lax.rsqrt(jnp.sum(s, axis=1, keepdims=True) + 1e-12)
        return dc * s * dc.T

    pa = pmat(aa_ref)
    pt = pmat(at_ref)
    pv = pmat(av_ref)
    qa_ref[...] = (pt + pv).astype(jnp.bfloat16)
    qt_ref[...] = (pv + pa).astype(jnp.bfloat16)
    qv_ref[...] = (pa + pt).astype(jnp.bfloat16)


def _fold_kernel(w_ref, d_ref, wb_ref, db_ref, m_ref, b2_ref):
    dbf = d_ref[...].astype(jnp.bfloat16)
    m_ref[...] = lax.dot_general(
        w_ref[...].astype(jnp.bfloat16), dbf, (((0,), (1,)), ((), ())),
        preferred_element_type=jnp.float32).astype(jnp.bfloat16)
    b2_ref[...] = lax.dot_general(
        wb_ref[...].astype(jnp.bfloat16), dbf, (((1,), (1,)), ((), ())),
        preferred_element_type=jnp.float32) + db_ref[...]


def _headmix_kernel(e_ref, m_ref, b2_ref, q_ref, o_ref):
    y = lax.dot_general(e_ref[...], m_ref[...], (((1,), (0,)), ((), ())),
                        preferred_element_type=jnp.float32)
    mixed = lax.dot_general(q_ref[...], y.astype(jnp.bfloat16),
                            (((1,), (0,)), ((), ())),
                            preferred_element_type=jnp.float32)
    o_ref[...] = 0.5 * y + 0.25 * mixed + b2_ref[...]


def kernel(a, t, v, mask, Wa_w, Wa_b, Wt_w, Wt_b, Wv_w, Wv_b,
           wa_w, wa_b, wt_w, wt_b, wv_w, wv_b,
           da_w, da_b, dt_w, dt_b, dv_w, dv_b):
    f32 = jnp.float32
    bf16 = jnp.bfloat16
    mrow = mask.reshape(1, N)
    mcol = mask.reshape(N, 1)
    ab = a.astype(bf16)
    tb = t.astype(bf16)
    vb = v.astype(bf16)

    full = lambda shape: pl.BlockSpec(shape, lambda i: (0, 0))
    ncols = lambda b: pl.BlockSpec((N, b), lambda i: (0, i))
    rows = lambda b, w: pl.BlockSpec((b, w), lambda i: (i, 0))

    enc_out = pl.pallas_call(
        _encode_kernel,
        grid=(ED // EB,),
        in_specs=[full((N, 1024)), full((N, 768)), full((N, 512)),
                  rows(EB, 1024), pl.BlockSpec((1, EB), lambda i: (0, i)),
                  rows(EB, 768), pl.BlockSpec((1, EB), lambda i: (0, i)),
                  rows(EB, 512), pl.BlockSpec((1, EB), lambda i: (0, i))],
        out_specs=[ncols(EB)] * 3 + [full((N, 1))] * 3,
        out_shape=[jax.ShapeDtypeStruct((N, ED), bf16)] * 3
        + [jax.ShapeDtypeStruct((N, 1), f32)] * 3,
    )
    eab, etb, evb, sqa, sqt, sqv = enc_out(
        ab, tb, vb, Wa_w, Wa_b.reshape(1, -1), Wt_w, Wt_b.reshape(1, -1),
        Wv_w, Wv_b.reshape(1, -1))

    aa, at, av = pl.pallas_call(
        _affinity_kernel,
        grid=(N // RB,),
        in_specs=[full((N, ED))] * 3 + [full((N, 1))] * 3
        + [full((1, N)), full((N, 1))],
        out_specs=[rows(RB, N)] * 3,
        out_shape=[jax.ShapeDtypeStruct((N, N), bf16)] * 3,
    )(eab, etb, evb, sqa, sqt, sqv, mrow, mcol)

    qa, qt, qv = pl.pallas_call(
        _symnorm_kernel,
        out_shape=[jax.ShapeDtypeStruct((N, N), bf16)] * 3,
    )(aa, at, av)

    def fold(w, wb, d, db):
        dout = d.shape[0]
        return pl.pallas_call(
            _fold_kernel,
            out_shape=[jax.ShapeDtypeStruct((ED, dout), bf16),
                       jax.ShapeDtypeStruct((1, dout), f32)],
        )(w, d, wb.reshape(1, -1), db.reshape(1, -1))

    ma, b2a = fold(wa_w, wa_b, da_w, da_b)
    mt, b2t = fold(wt_w, wt_b, dt_w, dt_b)
    mv, b2v = fold(wv_w, wv_b, dv_w, dv_b)

    def headmix(eb, m, b2, q):
        dout = m.shape[1]
        return pl.pallas_call(
            _headmix_kernel,
            grid=(dout // DB,),
            in_specs=[full((N, ED)), pl.BlockSpec((ED, DB), lambda i: (0, i)),
                      pl.BlockSpec((1, DB), lambda i: (0, i)),
                      full((N, N))],
            out_specs=ncols(DB),
            out_shape=jax.ShapeDtypeStruct((N, dout), f32),
        )(eb, m, b2, q)

    ra = headmix(eab, ma, b2a, qa)
    rt = headmix(etb, mt, b2t, qt)
    rv = headmix(evb, mv, b2v, qv)
    return jnp.concatenate([ra, rt, rv], axis=1)
